# baseline (speedup 1.0000x reference)
import functools

import jax
import jax.numpy as jnp
from jax import lax
from jax.experimental import pallas as pl
from jax.experimental.pallas import tpu as pltpu
from jax.experimental.pallas import tpu_sc as plsc

_ND = 2000
_NCELL = 1500
_B = 16384

_NG = 12
_NSEC = 2
_SEC = _B // _NSEC
_LANES = 16
_TAIL = 1408
_CAP = _SEC + _LANES


def _make_sc_kernel():
    mesh = plsc.VectorSubcoreMesh(core_axis_name="c", subcore_axis_name="s")

    @functools.partial(
        pl.kernel,
        mesh=mesh,
        compiler_params=pltpu.CompilerParams(needs_layout_passes=False),
        out_type=jax.ShapeDtypeStruct((_B,), jnp.float32),
        scratch_types=[
            pltpu.VMEM((_SEC,), jnp.int32),
            pltpu.VMEM((_SEC,), jnp.int32),
            pltpu.VMEM((_SEC,), jnp.float32),
            pltpu.VMEM((_CAP,), jnp.int32),
            pltpu.VMEM((_LANES, 128), jnp.float32),
            pltpu.VMEM((_LANES, 128), jnp.float32),
            pltpu.VMEM((_LANES, 128), jnp.float32),
            pltpu.VMEM((_LANES,), jnp.float32),
            pltpu.SemaphoreType.DMA,
        ],
    )
    def sck(x_hbm, did_hbm, cid_hbm, b_hbm, e_hbm, d_hbm,
            bt_hbm, et_hbm, dt_hbm, out_hbm,
            didv, cidv, xv, listv, bbuf, ebuf, dbuf, resv, sem):
        wid = lax.axis_index("s") * 2 + lax.axis_index("c")
        grp = wid % _NG
        sec = wid // _NG
        base = sec * _SEC
        col0 = grp * 128
        iota = jnp.arange(_LANES, dtype=jnp.int32)

        @pl.when(wid < _NG * _NSEC)
        def _():
            pltpu.sync_copy(did_hbm.at[pl.ds(base, _SEC)], didv)
            pltpu.sync_copy(cid_hbm.at[pl.ds(base, _SEC)], cidv)
            pltpu.sync_copy(x_hbm.at[pl.ds(base, _SEC)], xv)

            def scan_body(i, off):
                dv = didv[pl.ds(i * _LANES, _LANES)]
                cv = cidv[pl.ds(i * _LANES, _LANES)]
                m = lax.shift_right_logical(cv, 7) == grp
                c_loc = cv - col0
                packed = (dv << 20) | (c_loc << 13) | (i * _LANES + iota)
                plsc.store_compressed(
                    listv.at[pl.ds(off, _LANES)], packed, mask=m
                )
                return off + jnp.sum(m.astype(jnp.int32))

            count = lax.fori_loop(0, _SEC // _LANES, scan_body, jnp.int32(0))
            listv[pl.ds(count, _LANES)] = jnp.full((_LANES,), -1, jnp.int32)
            n_chunks = lax.shift_right_logical(count + (_LANES - 1), 4)

            def gather_loop(tb, te, td, cstart):
                def body(j, carry):
                    packed = listv[pl.ds(j * _LANES, _LANES)]
                    valid = packed >= 0
                    pos_l = packed & 0x1FFF
                    c_loc = lax.shift_right_logical(packed, 13) & 0x7F
                    drug = jnp.where(
                        valid, lax.shift_right_logical(packed, 20), 0
                    )
                    cb = pltpu.async_copy(
                        tb.at[drug, pl.ds(cstart, 128)], bbuf, sem
                    )
                    ce = pltpu.async_copy(
                        te.at[drug, pl.ds(cstart, 128)], ebuf, sem
                    )
                    cd = pltpu.async_copy(
                        td.at[drug, pl.ds(cstart, 128)], dbuf, sem
                    )
                    cb.wait()
                    ce.wait()
                    cd.wait()
                    xvals = plsc.load_gather(xv, [pos_l])
                    bb = plsc.load_gather(bbuf, [iota, c_loc])
                    ee = plsc.load_gather(ebuf, [iota, c_loc])
                    dd = plsc.load_gather(dbuf, [iota, c_loc])
                    t = bb * (xvals + ee)
                    resv[...] = dd / (1.0 + jnp.exp(-t))
                    posg = jnp.where(valid, base + pos_l, -1)
                    pltpu.async_copy(
                        resv,
                        out_hbm.at[plsc.Indices(posg, ignored_value=-1)],
                        sem,
                    ).wait()
                    return carry

                lax.fori_loop(0, n_chunks, body, jnp.int32(0))

            for gg in range(_NG - 1):
                @pl.when(grp == gg)
                def _(gg=gg):
                    gather_loop(b_hbm, e_hbm, d_hbm, 128 * gg)

            @pl.when(grp == _NG - 1)
            def _():
                gather_loop(bt_hbm, et_hbm, dt_hbm, 0)

    return sck


_sck = _make_sc_kernel()


@jax.jit
def kernel(x, drug_id, cell_id, b, e, d):
    pad = ((0, 0), (0, 128 - (_NCELL - _TAIL)))
    bt = jnp.pad(b[:, _TAIL:], pad)
    et = jnp.pad(e[:, _TAIL:], pad)
    dt = jnp.pad(d[:, _TAIL:], pad)
    return _sck(x, drug_id.astype(jnp.int32), cell_id.astype(jnp.int32),
                b, e, d, bt, et, dt)


# speedup vs baseline: 1.0475x; 1.0475x over previous
import functools

import jax
import jax.numpy as jnp
from jax import lax
from jax.experimental import pallas as pl
from jax.experimental.pallas import tpu as pltpu
from jax.experimental.pallas import tpu_sc as plsc

_ND = 2000
_NCELL = 1500
_B = 16384

_NG = 12
_NSEC = 2
_SEC = _B // _NSEC
_LANES = 16
_C = 64
_TAIL = 1408
_CAP = _SEC + _C


def _make_sc_kernel():
    mesh = plsc.VectorSubcoreMesh(core_axis_name="c", subcore_axis_name="s")

    @functools.partial(
        pl.kernel,
        mesh=mesh,
        compiler_params=pltpu.CompilerParams(needs_layout_passes=False),
        out_type=jax.ShapeDtypeStruct((_B,), jnp.float32),
        scratch_types=[
            pltpu.VMEM((_SEC,), jnp.int32),
            pltpu.VMEM((_SEC,), jnp.int32),
            pltpu.VMEM((_SEC,), jnp.float32),
            pltpu.VMEM((_CAP,), jnp.int32),
            pltpu.VMEM((2, _C, 128), jnp.float32),
            pltpu.VMEM((2, _C, 128), jnp.float32),
            pltpu.VMEM((2, _C, 128), jnp.float32),
            pltpu.VMEM((2, _C), jnp.int32),
            pltpu.VMEM((2, _C), jnp.float32),
            pltpu.VMEM((2, _C), jnp.int32),
            pltpu.SemaphoreType.DMA,
            pltpu.SemaphoreType.DMA,
        ],
    )
    def sck(x_hbm, did_hbm, cid_hbm, b_hbm, e_hbm, d_hbm,
            bt_hbm, et_hbm, dt_hbm, out_hbm,
            didv, cidv, xv, listv, bbuf, ebuf, dbuf,
            drugix, resv, possc, sem, sem2):
        wid = lax.axis_index("s") * 2 + lax.axis_index("c")
        grp = wid % _NG
        sec = wid // _NG
        base = sec * _SEC
        col0 = grp * 128
        iota = jnp.arange(_LANES, dtype=jnp.int32)

        @pl.when(wid < _NG * _NSEC)
        def _():
            pltpu.sync_copy(did_hbm.at[pl.ds(base, _SEC)], didv)
            pltpu.sync_copy(cid_hbm.at[pl.ds(base, _SEC)], cidv)
            pltpu.sync_copy(x_hbm.at[pl.ds(base, _SEC)], xv)

            def scan_body(i, off):
                b0 = i * _C
                ms, packs, cnts = [], [], []
                for k in range(4):
                    dv = didv[pl.ds(b0 + k * _LANES, _LANES)]
                    cv = cidv[pl.ds(b0 + k * _LANES, _LANES)]
                    m = lax.shift_right_logical(cv, 7) == grp
                    packed = (
                        (dv << 20) | ((cv - col0) << 13)
                        | (b0 + k * _LANES + iota)
                    )
                    ms.append(m)
                    packs.append(packed)
                    cnts.append(jnp.sum(m.astype(jnp.int32)))
                o = off
                for k in range(4):
                    plsc.store_compressed(
                        listv.at[pl.ds(o, _LANES)], packs[k], mask=ms[k]
                    )
                    o = o + cnts[k]
                return o

            count = lax.fori_loop(0, _SEC // _C, scan_body, jnp.int32(0))
            for k in range(4):
                listv[pl.ds(count + k * _LANES, _LANES)] = jnp.full(
                    (_LANES,), -1, jnp.int32
                )
            n_chunks = lax.shift_right_logical(count + (_C - 1), 6)

            def gather_loop(tb, te, td, cstart):
                def prep(t, slot):
                    for k in range(4):
                        packed = listv[pl.ds(t * _C + k * _LANES, _LANES)]
                        drug = jnp.where(
                            packed >= 0,
                            lax.shift_right_logical(packed, 20),
                            0,
                        )
                        drugix[slot, pl.ds(k * _LANES, _LANES)] = drug
                    di = drugix.at[slot]
                    pltpu.async_copy(
                        tb.at[di, pl.ds(cstart, 128)], bbuf.at[slot], sem
                    )
                    pltpu.async_copy(
                        te.at[di, pl.ds(cstart, 128)], ebuf.at[slot], sem
                    )
                    pltpu.async_copy(
                        td.at[di, pl.ds(cstart, 128)], dbuf.at[slot], sem
                    )

                @pl.when(n_chunks > 0)
                def _():
                    prep(jnp.int32(0), jnp.int32(0))

                    def body(t, carry):
                        p = t & 1

                        @pl.when(t + 1 < n_chunks)
                        def _():
                            prep(t + 1, 1 - p)

                        di = drugix.at[p]
                        pltpu.make_async_copy(
                            tb.at[di, pl.ds(cstart, 128)], bbuf.at[p], sem
                        ).wait()
                        pltpu.make_async_copy(
                            te.at[di, pl.ds(cstart, 128)], ebuf.at[p], sem
                        ).wait()
                        pltpu.make_async_copy(
                            td.at[di, pl.ds(cstart, 128)], dbuf.at[p], sem
                        ).wait()
                        psplat = jnp.full((_LANES,), p, jnp.int32)
                        for k in range(4):
                            packed = listv[pl.ds(t * _C + k * _LANES, _LANES)]
                            valid = packed >= 0
                            pos_l = packed & 0x1FFF
                            c_loc = lax.shift_right_logical(packed, 13) & 0x7F
                            rows = k * _LANES + iota
                            xvals = plsc.load_gather(xv, [pos_l])
                            bb = plsc.load_gather(bbuf, [psplat, rows, c_loc])
                            ee = plsc.load_gather(ebuf, [psplat, rows, c_loc])
                            dd = plsc.load_gather(dbuf, [psplat, rows, c_loc])
                            tt = bb * (xvals + ee)
                            resv[p, pl.ds(k * _LANES, _LANES)] = (
                                dd / (1.0 + jnp.exp(-tt))
                            )
                            possc[p, pl.ds(k * _LANES, _LANES)] = jnp.where(
                                valid, base + pos_l, -1
                            )
                        pltpu.async_copy(
                            resv.at[p],
                            out_hbm.at[
                                plsc.Indices(possc.at[p], ignored_value=-1)
                            ],
                            sem2,
                        ).wait()
                        return carry

                    lax.fori_loop(0, n_chunks, body, jnp.int32(0))

            @pl.when(grp < _NG - 1)
            def _():
                gather_loop(b_hbm, e_hbm, d_hbm, col0)

            @pl.when(grp == _NG - 1)
            def _():
                gather_loop(bt_hbm, et_hbm, dt_hbm, 0)

    return sck


_sck = _make_sc_kernel()


@jax.jit
def kernel(x, drug_id, cell_id, b, e, d):
    pad = ((0, 0), (0, 128 - (_NCELL - _TAIL)))
    bt = jnp.pad(b[:, _TAIL:], pad)
    et = jnp.pad(e[:, _TAIL:], pad)
    dt = jnp.pad(d[:, _TAIL:], pad)
    return _sck(x, drug_id.astype(jnp.int32), cell_id.astype(jnp.int32),
                b, e, d, bt, et, dt)


# speedup vs baseline: 1.9280x; 1.8406x over previous
import functools

import jax
import jax.numpy as jnp
from jax import lax
from jax.experimental import pallas as pl
from jax.experimental.pallas import tpu as pltpu
from jax.experimental.pallas import tpu_sc as plsc

_ND = 2000
_NCELL = 1500
_B = 16384

_NG = 12
_SEC = _B // 2
_LANES = 16
_C = 64
_TAIL = 1408
_CAP = _SEC + _C
_SLICE = _SEC // 16


def _make_sc_kernel():
    mesh = plsc.VectorSubcoreMesh(core_axis_name="c", subcore_axis_name="s")

    @functools.partial(
        pl.kernel,
        mesh=mesh,
        compiler_params=pltpu.CompilerParams(needs_layout_passes=False),
        out_type=jax.ShapeDtypeStruct((_B,), jnp.float32),
        scratch_types=[
            pltpu.VMEM((_SEC,), jnp.int32),
            pltpu.VMEM((_SEC,), jnp.int32),
            pltpu.VMEM((_SEC,), jnp.float32),
            pltpu.VMEM((_CAP,), jnp.int32),
            pltpu.VMEM((_C, 128), jnp.float32),
            pltpu.VMEM((_C, 128), jnp.float32),
            pltpu.VMEM((_C, 128), jnp.float32),
            pltpu.VMEM((_C,), jnp.int32),
            pltpu.VMEM((_C,), jnp.float32),
            pltpu.VMEM((_C,), jnp.int32),
            pltpu.VMEM((_SLICE,), jnp.float32),
            pltpu.VMEM_SHARED((_SEC,), jnp.float32),
            pltpu.SemaphoreType.DMA,
            pltpu.SemaphoreType.DMA,
        ],
    )
    def sck(x_hbm, did_hbm, cid_hbm, b_hbm, e_hbm, d_hbm,
            bt_hbm, et_hbm, dt_hbm, out_hbm,
            didv, cidv, xv, listv, bbuf, ebuf, dbuf,
            drugix, resv, possc, outstage, shared, sem, sem2):
        grp = lax.axis_index("s")
        sec = lax.axis_index("c")
        base = sec * _SEC
        col0 = grp * 128
        iota = jnp.arange(_LANES, dtype=jnp.int32)

        @pl.when(grp < _NG)
        def _():
            pltpu.sync_copy(did_hbm.at[pl.ds(base, _SEC)], didv)
            pltpu.sync_copy(cid_hbm.at[pl.ds(base, _SEC)], cidv)
            pltpu.sync_copy(x_hbm.at[pl.ds(base, _SEC)], xv)

            def scan_body(i, off):
                b0 = i * _C
                ms, packs, cnts = [], [], []
                for k in range(4):
                    dv = didv[pl.ds(b0 + k * _LANES, _LANES)]
                    cv = cidv[pl.ds(b0 + k * _LANES, _LANES)]
                    m = lax.shift_right_logical(cv, 7) == grp
                    packed = (
                        (dv << 20) | ((cv - col0) << 13)
                        | (b0 + k * _LANES + iota)
                    )
                    ms.append(m)
                    packs.append(packed)
                    cnts.append(jnp.sum(m.astype(jnp.int32)))
                o = off
                for k in range(4):
                    plsc.store_compressed(
                        listv.at[pl.ds(o, _LANES)], packs[k], mask=ms[k]
                    )
                    o = o + cnts[k]
                return o

            count = lax.fori_loop(0, _SEC // _C, scan_body, jnp.int32(0))
            for k in range(4):
                listv[pl.ds(count + k * _LANES, _LANES)] = jnp.full(
                    (_LANES,), -1, jnp.int32
                )
            n_chunks = lax.shift_right_logical(count + (_C - 1), 6)

            def gather_loop(tb, te, td, cstart):
                def body(t, carry):
                    for k in range(4):
                        packed = listv[pl.ds(t * _C + k * _LANES, _LANES)]
                        drug = jnp.where(
                            packed >= 0,
                            lax.shift_right_logical(packed, 20),
                            0,
                        )
                        drugix[pl.ds(k * _LANES, _LANES)] = drug
                    di = drugix.at[...]
                    cb = pltpu.async_copy(
                        tb.at[di, pl.ds(cstart, 128)], bbuf, sem
                    )
                    ce = pltpu.async_copy(
                        te.at[di, pl.ds(cstart, 128)], ebuf, sem
                    )
                    cd = pltpu.async_copy(
                        td.at[di, pl.ds(cstart, 128)], dbuf, sem
                    )
                    cb.wait()
                    ce.wait()
                    cd.wait()
                    for k in range(4):
                        packed = listv[pl.ds(t * _C + k * _LANES, _LANES)]
                        valid = packed >= 0
                        pos_l = packed & 0x1FFF
                        c_loc = lax.shift_right_logical(packed, 13) & 0x7F
                        rows = k * _LANES + iota
                        xvals = plsc.load_gather(xv, [pos_l])
                        bb = plsc.load_gather(bbuf, [rows, c_loc])
                        ee = plsc.load_gather(ebuf, [rows, c_loc])
                        dd = plsc.load_gather(dbuf, [rows, c_loc])
                        tt = bb * (xvals + ee)
                        resv[pl.ds(k * _LANES, _LANES)] = (
                            dd / (1.0 + jnp.exp(-tt))
                        )
                        possc[pl.ds(k * _LANES, _LANES)] = jnp.where(
                            valid, pos_l, -1
                        )
                    pltpu.async_copy(
                        resv,
                        shared.at[plsc.Indices(possc, ignored_value=-1)],
                        sem2,
                    ).wait()
                    return carry

                lax.fori_loop(0, n_chunks, body, jnp.int32(0))

            @pl.when(grp < _NG - 1)
            def _():
                gather_loop(b_hbm, e_hbm, d_hbm, col0)

            @pl.when(grp == _NG - 1)
            def _():
                gather_loop(bt_hbm, et_hbm, dt_hbm, 0)

        plsc.subcore_barrier()
        pltpu.sync_copy(shared.at[pl.ds(grp * _SLICE, _SLICE)], outstage)
        pltpu.sync_copy(outstage, out_hbm.at[pl.ds(base + grp * _SLICE, _SLICE)])

    return sck


_sck = _make_sc_kernel()


@jax.jit
def kernel(x, drug_id, cell_id, b, e, d):
    pad = ((0, 0), (0, 128 - (_NCELL - _TAIL)))
    bt = jnp.pad(b[:, _TAIL:], pad)
    et = jnp.pad(e[:, _TAIL:], pad)
    dt = jnp.pad(d[:, _TAIL:], pad)
    return _sck(x, drug_id.astype(jnp.int32), cell_id.astype(jnp.int32),
                b, e, d, bt, et, dt)


# speedup vs baseline: 2.2301x; 1.1567x over previous
import functools

import jax
import jax.numpy as jnp
from jax import lax
from jax.experimental import pallas as pl
from jax.experimental.pallas import tpu as pltpu
from jax.experimental.pallas import tpu_sc as plsc

_ND = 2000
_NCELL = 1500
_B = 16384

_NG = 12
_SEC = _B // 2
_LANES = 16
_C = 64
_TAIL = 1408
_CAP = _SEC + _C
_SLICE = _SEC // 16


def _make_sc_kernel():
    mesh = plsc.VectorSubcoreMesh(core_axis_name="c", subcore_axis_name="s")

    @functools.partial(
        pl.kernel,
        mesh=mesh,
        compiler_params=pltpu.CompilerParams(needs_layout_passes=False),
        out_type=jax.ShapeDtypeStruct((_B,), jnp.float32),
        scratch_types=[
            pltpu.VMEM((_SEC,), jnp.int32),
            pltpu.VMEM((_SEC,), jnp.int32),
            pltpu.VMEM((_SEC,), jnp.float32),
            pltpu.VMEM((_CAP,), jnp.int32),
            pltpu.VMEM((_C, 128), jnp.float32),
            pltpu.VMEM((_C, 128), jnp.float32),
            pltpu.VMEM((_C, 128), jnp.float32),
            pltpu.VMEM((_C, 128), jnp.float32),
            pltpu.VMEM((_C, 128), jnp.float32),
            pltpu.VMEM((_C, 128), jnp.float32),
            pltpu.VMEM((_C,), jnp.int32),
            pltpu.VMEM((_C,), jnp.int32),
            pltpu.VMEM((_C,), jnp.float32),
            pltpu.VMEM((_C,), jnp.int32),
            pltpu.VMEM((_SLICE,), jnp.float32),
            pltpu.VMEM_SHARED((_SEC,), jnp.float32),
            pltpu.SemaphoreType.DMA,
            pltpu.SemaphoreType.DMA,
            pltpu.SemaphoreType.DMA,
        ],
    )
    def sck(x_hbm, did_hbm, cid_hbm, b_hbm, e_hbm, d_hbm,
            bt_hbm, et_hbm, dt_hbm, out_hbm,
            didv, cidv, xv, listv,
            bbufa, ebufa, dbufa, bbufb, ebufb, dbufb,
            dixa, dixb, resv, possc, outstage, shared,
            sem, sema, semb):
        grp = lax.axis_index("s")
        sec = lax.axis_index("c")
        base = sec * _SEC
        col0 = grp * 128
        iota = jnp.arange(_LANES, dtype=jnp.int32)

        @pl.when(grp < _NG)
        def _():
            c1 = pltpu.async_copy(did_hbm.at[pl.ds(base, _SEC)], didv, sem)
            c2 = pltpu.async_copy(cid_hbm.at[pl.ds(base, _SEC)], cidv, sem)
            c3 = pltpu.async_copy(x_hbm.at[pl.ds(base, _SEC)], xv, sem)
            c1.wait()
            c2.wait()
            c3.wait()

            def scan_body(i, off):
                b0 = i * _C
                ms, packs, cnts = [], [], []
                for k in range(4):
                    dv = didv[pl.ds(b0 + k * _LANES, _LANES)]
                    cv = cidv[pl.ds(b0 + k * _LANES, _LANES)]
                    m = lax.shift_right_logical(cv, 7) == grp
                    packed = (
                        (dv << 20) | ((cv - col0) << 13)
                        | (b0 + k * _LANES + iota)
                    )
                    ms.append(m)
                    packs.append(packed)
                    cnts.append(jnp.sum(m.astype(jnp.int32)))
                o = off
                for k in range(4):
                    plsc.store_compressed(
                        listv.at[pl.ds(o, _LANES)], packs[k], mask=ms[k]
                    )
                    o = o + cnts[k]
                return o

            count = lax.fori_loop(0, _SEC // _C, scan_body, jnp.int32(0))
            for k in range(4):
                listv[pl.ds(count + k * _LANES, _LANES)] = jnp.full(
                    (_LANES,), -1, jnp.int32
                )
            n_chunks = lax.shift_right_logical(count + (_C - 1), 6)

            def gather_loop(tb, te, td, cstart):
                slot_a = (bbufa, ebufa, dbufa, dixa, sema)
                slot_b = (bbufb, ebufb, dbufb, dixb, semb)

                def prepfire(t, slot):
                    bb, eb, db, dix, sm = slot
                    for k in range(4):
                        packed = listv[pl.ds(t * _C + k * _LANES, _LANES)]
                        drug = jnp.where(
                            packed >= 0,
                            lax.shift_right_logical(packed, 20),
                            0,
                        )
                        dix[pl.ds(k * _LANES, _LANES)] = drug
                    di = dix.at[...]
                    pltpu.async_copy(tb.at[di, pl.ds(cstart, 128)], bb, sm)
                    pltpu.async_copy(te.at[di, pl.ds(cstart, 128)], eb, sm)
                    pltpu.async_copy(td.at[di, pl.ds(cstart, 128)], db, sm)

                def consume(t, slot):
                    bb, eb, db, dix, sm = slot
                    di = dix.at[...]
                    pltpu.make_async_copy(
                        tb.at[di, pl.ds(cstart, 128)], bb, sm
                    ).wait()
                    pltpu.make_async_copy(
                        te.at[di, pl.ds(cstart, 128)], eb, sm
                    ).wait()
                    pltpu.make_async_copy(
                        td.at[di, pl.ds(cstart, 128)], db, sm
                    ).wait()
                    for k in range(4):
                        packed = listv[pl.ds(t * _C + k * _LANES, _LANES)]
                        valid = packed >= 0
                        pos_l = packed & 0x1FFF
                        c_loc = lax.shift_right_logical(packed, 13) & 0x7F
                        rows = k * _LANES + iota
                        xvals = plsc.load_gather(xv, [pos_l])
                        bv = plsc.load_gather(bb, [rows, c_loc])
                        ev = plsc.load_gather(eb, [rows, c_loc])
                        dv = plsc.load_gather(db, [rows, c_loc])
                        tt = bv * (xvals + ev)
                        resv[pl.ds(k * _LANES, _LANES)] = (
                            dv / (1.0 + jnp.exp(-tt))
                        )
                        possc[pl.ds(k * _LANES, _LANES)] = jnp.where(
                            valid, pos_l, -1
                        )
                    pltpu.async_copy(
                        resv,
                        shared.at[plsc.Indices(possc, ignored_value=-1)],
                        sem,
                    ).wait()

                @pl.when(n_chunks > 0)
                def _():
                    prepfire(jnp.int32(0), slot_a)

                    def body(t2, carry):
                        a = 2 * t2
                        bq = a + 1

                        @pl.when(bq < n_chunks)
                        def _():
                            prepfire(bq, slot_b)

                        consume(a, slot_a)

                        @pl.when(a + 2 < n_chunks)
                        def _():
                            prepfire(a + 2, slot_a)

                        @pl.when(bq < n_chunks)
                        def _():
                            consume(bq, slot_b)

                        return carry

                    n2 = lax.shift_right_logical(n_chunks + 1, 1)
                    lax.fori_loop(0, n2, body, jnp.int32(0))

            @pl.when(grp < _NG - 1)
            def _():
                gather_loop(b_hbm, e_hbm, d_hbm, col0)

            @pl.when(grp == _NG - 1)
            def _():
                gather_loop(bt_hbm, et_hbm, dt_hbm, 0)

        plsc.subcore_barrier()
        pltpu.sync_copy(shared.at[pl.ds(grp * _SLICE, _SLICE)], outstage)
        pltpu.sync_copy(outstage, out_hbm.at[pl.ds(base + grp * _SLICE, _SLICE)])

    return sck


_sck = _make_sc_kernel()


@jax.jit
def kernel(x, drug_id, cell_id, b, e, d):
    pad = ((0, 0), (0, 128 - (_NCELL - _TAIL)))
    bt = jnp.pad(b[:, _TAIL:], pad)
    et = jnp.pad(e[:, _TAIL:], pad)
    dt = jnp.pad(d[:, _TAIL:], pad)
    return _sck(x, drug_id.astype(jnp.int32), cell_id.astype(jnp.int32),
                b, e, d, bt, et, dt)


# speedup vs baseline: 2.2804x; 1.0226x over previous
import functools

import jax
import jax.numpy as jnp
from jax import lax
from jax.experimental import pallas as pl
from jax.experimental.pallas import tpu as pltpu
from jax.experimental.pallas import tpu_sc as plsc

_ND = 2000
_NCELL = 1500
_B = 16384

_NG = 12
_SEC = _B // 2
_LANES = 16
_C = 64
_TAIL = 1408
_CAP = _SEC + _C
_SLICE = _SEC // 16


def _make_sc_kernel():
    mesh = plsc.VectorSubcoreMesh(core_axis_name="c", subcore_axis_name="s")

    @functools.partial(
        pl.kernel,
        mesh=mesh,
        compiler_params=pltpu.CompilerParams(needs_layout_passes=False),
        out_type=jax.ShapeDtypeStruct((_B,), jnp.float32),
        scratch_types=[
            pltpu.VMEM((_SEC,), jnp.int32),
            pltpu.VMEM((_SEC,), jnp.int32),
            pltpu.VMEM((_SEC,), jnp.float32),
            pltpu.VMEM((_CAP,), jnp.int32),
            pltpu.VMEM((_C, 128), jnp.float32),
            pltpu.VMEM((_C, 128), jnp.float32),
            pltpu.VMEM((_C, 128), jnp.float32),
            pltpu.VMEM((_C, 128), jnp.float32),
            pltpu.VMEM((_C, 128), jnp.float32),
            pltpu.VMEM((_C, 128), jnp.float32),
            pltpu.VMEM((_C,), jnp.int32),
            pltpu.VMEM((_C,), jnp.int32),
            pltpu.VMEM((_C,), jnp.float32),
            pltpu.VMEM((_C,), jnp.int32),
            pltpu.VMEM((_SLICE,), jnp.float32),
            pltpu.VMEM_SHARED((_SEC,), jnp.float32),
            pltpu.SemaphoreType.DMA,
            pltpu.SemaphoreType.DMA,
            pltpu.SemaphoreType.DMA,
        ],
    )
    def sck(x_hbm, did_hbm, cid_hbm, b_hbm, e_hbm, d_hbm,
            bt_hbm, et_hbm, dt_hbm, out_hbm,
            didv, cidv, xv, listv,
            bbufa, ebufa, dbufa, bbufb, ebufb, dbufb,
            dixa, dixb, resv, possc, outstage, shared,
            sem, sema, semb):
        grp = lax.axis_index("s")
        sec = lax.axis_index("c")
        base = sec * _SEC
        col0 = grp * 128
        iota = jnp.arange(_LANES, dtype=jnp.int32)

        @pl.when(grp < _NG)
        def _():
            c1 = pltpu.async_copy(did_hbm.at[pl.ds(base, _SEC)], didv, sem)
            c2 = pltpu.async_copy(cid_hbm.at[pl.ds(base, _SEC)], cidv, sem)
            c3 = pltpu.async_copy(x_hbm.at[pl.ds(base, _SEC)], xv, sem)
            c1.wait()
            c2.wait()

            def scan_body(i, off):
                b0 = i * 2 * _C
                ms, packs, cnts = [], [], []
                for k in range(8):
                    dv = didv[pl.ds(b0 + k * _LANES, _LANES)]
                    cv = cidv[pl.ds(b0 + k * _LANES, _LANES)]
                    m = lax.shift_right_logical(cv, 7) == grp
                    packed = (
                        (dv << 20) | ((cv - col0) << 13)
                        | (b0 + k * _LANES + iota)
                    )
                    ms.append(m)
                    packs.append(packed)
                    cnts.append(jnp.sum(m.astype(jnp.int32)))
                o = off
                for k in range(8):
                    plsc.store_compressed(
                        listv.at[pl.ds(o, _LANES)], packs[k], mask=ms[k]
                    )
                    o = o + cnts[k]
                return o

            count = lax.fori_loop(0, _SEC // (2 * _C), scan_body, jnp.int32(0))
            c3.wait()
            for k in range(4):
                listv[pl.ds(count + k * _LANES, _LANES)] = jnp.full(
                    (_LANES,), -1, jnp.int32
                )
            n_chunks = lax.shift_right_logical(count + (_C - 1), 6)

            def gather_loop(tb, te, td, cstart):
                slot_a = (bbufa, ebufa, dbufa, dixa, sema)
                slot_b = (bbufb, ebufb, dbufb, dixb, semb)

                def prepfire(t, slot):
                    bb, eb, db, dix, sm = slot
                    for k in range(4):
                        packed = listv[pl.ds(t * _C + k * _LANES, _LANES)]
                        drug = jnp.where(
                            packed >= 0,
                            lax.shift_right_logical(packed, 20),
                            0,
                        )
                        dix[pl.ds(k * _LANES, _LANES)] = drug
                    di = dix.at[...]
                    pltpu.async_copy(tb.at[di, pl.ds(cstart, 128)], bb, sm)
                    pltpu.async_copy(te.at[di, pl.ds(cstart, 128)], eb, sm)
                    pltpu.async_copy(td.at[di, pl.ds(cstart, 128)], db, sm)

                def consume(t, slot):
                    bb, eb, db, dix, sm = slot
                    di = dix.at[...]
                    pltpu.make_async_copy(
                        tb.at[di, pl.ds(cstart, 128)], bb, sm
                    ).wait()
                    pltpu.make_async_copy(
                        te.at[di, pl.ds(cstart, 128)], eb, sm
                    ).wait()
                    pltpu.make_async_copy(
                        td.at[di, pl.ds(cstart, 128)], db, sm
                    ).wait()
                    for k in range(4):
                        packed = listv[pl.ds(t * _C + k * _LANES, _LANES)]
                        valid = packed >= 0
                        pos_l = packed & 0x1FFF
                        c_loc = lax.shift_right_logical(packed, 13) & 0x7F
                        rows = k * _LANES + iota
                        xvals = plsc.load_gather(xv, [pos_l])
                        bv = plsc.load_gather(bb, [rows, c_loc])
                        ev = plsc.load_gather(eb, [rows, c_loc])
                        dv = plsc.load_gather(db, [rows, c_loc])
                        tt = bv * (xvals + ev)
                        resv[pl.ds(k * _LANES, _LANES)] = (
                            dv / (1.0 + jnp.exp(-tt))
                        )
                        possc[pl.ds(k * _LANES, _LANES)] = jnp.where(
                            valid, pos_l, -1
                        )
                    pltpu.async_copy(
                        resv,
                        shared.at[plsc.Indices(possc, ignored_value=-1)],
                        sem,
                    ).wait()

                @pl.when(n_chunks > 0)
                def _():
                    prepfire(jnp.int32(0), slot_a)

                    def body(t2, carry):
                        a = 2 * t2
                        bq = a + 1

                        @pl.when(bq < n_chunks)
                        def _():
                            prepfire(bq, slot_b)

                        consume(a, slot_a)

                        @pl.when(a + 2 < n_chunks)
                        def _():
                            prepfire(a + 2, slot_a)

                        @pl.when(bq < n_chunks)
                        def _():
                            consume(bq, slot_b)

                        return carry

                    n2 = lax.shift_right_logical(n_chunks + 1, 1)
                    lax.fori_loop(0, n2, body, jnp.int32(0))

            @pl.when(grp < _NG - 1)
            def _():
                gather_loop(b_hbm, e_hbm, d_hbm, col0)

            @pl.when(grp == _NG - 1)
            def _():
                gather_loop(bt_hbm, et_hbm, dt_hbm, 0)

        plsc.subcore_barrier()
        pltpu.sync_copy(shared.at[pl.ds(grp * _SLICE, _SLICE)], outstage)
        pltpu.sync_copy(outstage, out_hbm.at[pl.ds(base + grp * _SLICE, _SLICE)])

    return sck


_sck = _make_sc_kernel()


@jax.jit
def kernel(x, drug_id, cell_id, b, e, d):
    pad = ((0, 0), (0, 128 - (_NCELL - _TAIL)))
    bt = jnp.pad(b[:, _TAIL:], pad)
    et = jnp.pad(e[:, _TAIL:], pad)
    dt = jnp.pad(d[:, _TAIL:], pad)
    return _sck(x, drug_id.astype(jnp.int32), cell_id.astype(jnp.int32),
                b, e, d, bt, et, dt)


# speedup vs baseline: 2.3066x; 1.0115x over previous
import functools

import jax
import jax.numpy as jnp
from jax import lax
from jax.experimental import pallas as pl
from jax.experimental.pallas import tpu as pltpu
from jax.experimental.pallas import tpu_sc as plsc

_ND = 2000
_NCELL = 1500
_B = 16384

_NG = 12
_SEC = _B // 2
_LANES = 16
_C = 64
_TAIL = 1408
_CAP = _SEC + _C
_SLICE = _SEC // 16


def _make_sc_kernel():
    mesh = plsc.VectorSubcoreMesh(core_axis_name="c", subcore_axis_name="s")

    @functools.partial(
        pl.kernel,
        mesh=mesh,
        compiler_params=pltpu.CompilerParams(needs_layout_passes=False),
        out_type=jax.ShapeDtypeStruct((_B,), jnp.float32),
        scratch_types=[
            pltpu.VMEM((_SEC,), jnp.int32),
            pltpu.VMEM((_SEC,), jnp.int32),
            pltpu.VMEM((_SEC,), jnp.float32),
            pltpu.VMEM((_CAP,), jnp.int32),
            pltpu.VMEM((_C, 128), jnp.float32),
            pltpu.VMEM((_C, 128), jnp.float32),
            pltpu.VMEM((_C, 128), jnp.float32),
            pltpu.VMEM((_C, 128), jnp.float32),
            pltpu.VMEM((_C, 128), jnp.float32),
            pltpu.VMEM((_C, 128), jnp.float32),
            pltpu.VMEM((_C,), jnp.int32),
            pltpu.VMEM((_C,), jnp.int32),
            pltpu.VMEM((_C,), jnp.float32),
            pltpu.VMEM((_C,), jnp.int32),
            pltpu.VMEM((_C,), jnp.float32),
            pltpu.VMEM((_C,), jnp.int32),
            pltpu.VMEM((_SLICE,), jnp.float32),
            pltpu.VMEM_SHARED((_SEC,), jnp.float32),
            pltpu.SemaphoreType.DMA,
            pltpu.SemaphoreType.DMA,
            pltpu.SemaphoreType.DMA,
            pltpu.SemaphoreType.DMA,
            pltpu.SemaphoreType.DMA,
        ],
    )
    def sck(x_hbm, did_hbm, cid_hbm, b_hbm, e_hbm, d_hbm,
            bt_hbm, et_hbm, dt_hbm, out_hbm,
            didv, cidv, xv, listv,
            bbufa, ebufa, dbufa, bbufb, ebufb, dbufb,
            dixa, dixb, resva, possca, resvb, posscb, outstage, shared,
            sem, sema, semb, semsca, semscb):
        grp = lax.axis_index("s")
        sec = lax.axis_index("c")
        base = sec * _SEC
        col0 = grp * 128
        iota = jnp.arange(_LANES, dtype=jnp.int32)

        @pl.when(grp < _NG)
        def _():
            c1 = pltpu.async_copy(did_hbm.at[pl.ds(base, _SEC)], didv, sem)
            c2 = pltpu.async_copy(cid_hbm.at[pl.ds(base, _SEC)], cidv, sem)
            c3 = pltpu.async_copy(x_hbm.at[pl.ds(base, _SEC)], xv, sem)
            c1.wait()
            c2.wait()

            def scan_body(i, off):
                b0 = i * 2 * _C
                ms, packs, cnts = [], [], []
                for k in range(8):
                    dv = didv[pl.ds(b0 + k * _LANES, _LANES)]
                    cv = cidv[pl.ds(b0 + k * _LANES, _LANES)]
                    m = lax.shift_right_logical(cv, 7) == grp
                    packed = (
                        (dv << 20) | ((cv - col0) << 13)
                        | (b0 + k * _LANES + iota)
                    )
                    ms.append(m)
                    packs.append(packed)
                    cnts.append(jnp.sum(m.astype(jnp.int32)))
                o = off
                for k in range(8):
                    plsc.store_compressed(
                        listv.at[pl.ds(o, _LANES)], packs[k], mask=ms[k]
                    )
                    o = o + cnts[k]
                return o

            count = lax.fori_loop(0, _SEC // (2 * _C), scan_body, jnp.int32(0))
            c3.wait()
            for k in range(4):
                listv[pl.ds(count + k * _LANES, _LANES)] = jnp.full(
                    (_LANES,), -1, jnp.int32
                )
            n_chunks = lax.shift_right_logical(count + (_C - 1), 6)

            def gather_loop(tb, te, td, cstart):
                slot_a = (bbufa, ebufa, dbufa, dixa, sema, resva, possca, semsca)
                slot_b = (bbufb, ebufb, dbufb, dixb, semb, resvb, posscb, semscb)

                def prepfire(t, slot):
                    bb, eb, db, dix, sm = slot[:5]
                    for k in range(4):
                        packed = listv[pl.ds(t * _C + k * _LANES, _LANES)]
                        drug = jnp.where(
                            packed >= 0,
                            lax.shift_right_logical(packed, 20),
                            0,
                        )
                        dix[pl.ds(k * _LANES, _LANES)] = drug
                    di = dix.at[...]
                    pltpu.async_copy(tb.at[di, pl.ds(cstart, 128)], bb, sm)
                    pltpu.async_copy(te.at[di, pl.ds(cstart, 128)], eb, sm)
                    pltpu.async_copy(td.at[di, pl.ds(cstart, 128)], db, sm)

                def scatter_wait(slot):
                    _, _, _, _, _, resv, possc, smsc = slot
                    pltpu.make_async_copy(
                        resv,
                        shared.at[plsc.Indices(possc, ignored_value=-1)],
                        smsc,
                    ).wait()

                def consume(t, slot):
                    bb, eb, db, dix, sm, resv, possc, smsc = slot
                    @pl.when(t >= 2)
                    def _():
                        scatter_wait(slot)
                    di = dix.at[...]
                    pltpu.make_async_copy(
                        tb.at[di, pl.ds(cstart, 128)], bb, sm
                    ).wait()
                    pltpu.make_async_copy(
                        te.at[di, pl.ds(cstart, 128)], eb, sm
                    ).wait()
                    pltpu.make_async_copy(
                        td.at[di, pl.ds(cstart, 128)], db, sm
                    ).wait()
                    for k in range(4):
                        packed = listv[pl.ds(t * _C + k * _LANES, _LANES)]
                        valid = packed >= 0
                        pos_l = packed & 0x1FFF
                        c_loc = lax.shift_right_logical(packed, 13) & 0x7F
                        rows = k * _LANES + iota
                        xvals = plsc.load_gather(xv, [pos_l])
                        bv = plsc.load_gather(bb, [rows, c_loc])
                        ev = plsc.load_gather(eb, [rows, c_loc])
                        dv = plsc.load_gather(db, [rows, c_loc])
                        tt = bv * (xvals + ev)
                        resv[pl.ds(k * _LANES, _LANES)] = (
                            dv / (1.0 + jnp.exp(-tt))
                        )
                        possc[pl.ds(k * _LANES, _LANES)] = jnp.where(
                            valid, pos_l, -1
                        )
                    pltpu.async_copy(
                        resv,
                        shared.at[plsc.Indices(possc, ignored_value=-1)],
                        smsc,
                    )

                @pl.when(n_chunks > 0)
                def _():
                    prepfire(jnp.int32(0), slot_a)

                    def body(t2, carry):
                        a = 2 * t2
                        bq = a + 1

                        @pl.when(bq < n_chunks)
                        def _():
                            prepfire(bq, slot_b)

                        consume(a, slot_a)

                        @pl.when(a + 2 < n_chunks)
                        def _():
                            prepfire(a + 2, slot_a)

                        @pl.when(bq < n_chunks)
                        def _():
                            consume(bq, slot_b)

                        return carry

                    n2 = lax.shift_right_logical(n_chunks + 1, 1)
                    lax.fori_loop(0, n2, body, jnp.int32(0))
                    scatter_wait(slot_a)

                    @pl.when(n_chunks > 1)
                    def _():
                        scatter_wait(slot_b)

            @pl.when(grp < _NG - 1)
            def _():
                gather_loop(b_hbm, e_hbm, d_hbm, col0)

            @pl.when(grp == _NG - 1)
            def _():
                gather_loop(bt_hbm, et_hbm, dt_hbm, 0)

        plsc.subcore_barrier()
        pltpu.sync_copy(shared.at[pl.ds(grp * _SLICE, _SLICE)], outstage)
        pltpu.sync_copy(outstage, out_hbm.at[pl.ds(base + grp * _SLICE, _SLICE)])

    return sck


_sck = _make_sc_kernel()


@jax.jit
def kernel(x, drug_id, cell_id, b, e, d):
    pad = ((0, 0), (0, 128 - (_NCELL - _TAIL)))
    bt = jnp.pad(b[:, _TAIL:], pad)
    et = jnp.pad(e[:, _TAIL:], pad)
    dt = jnp.pad(d[:, _TAIL:], pad)
    return _sck(x, drug_id.astype(jnp.int32), cell_id.astype(jnp.int32),
                b, e, d, bt, et, dt)
